# baseline (speedup 1.0000x reference)
import math

import jax
import jax.numpy as jnp
from jax.experimental import pallas as pl
from jax.experimental.pallas import tpu as pltpu

NUM_TOKENS = 8192
INPUT_DIM = 1024
NUM_EXPERTS = 64
TOP_K = 8
CAPACITY = 2048
HIDDEN_DIM = 64

TILE = 256
NUM_TILES = NUM_TOKENS // TILE


def _moe_body(
    x_ref, gwt_ref, gb_ref, trile_ref, trilt_ref,
    w1c_ref, b1c_ref, w2sp_ref, expb_ref, out_ref, cnt_ref,
):
    i = pl.program_id(0)

    @pl.when(i == 0)
    def _():
        cnt_ref[...] = jnp.zeros_like(cnt_ref)

    E, H = NUM_EXPERTS, HIDDEN_DIM
    xf = x_ref[...]

    logits = (
        jnp.dot(xf, gwt_ref[...], preferred_element_type=jnp.float32)
        + gb_ref[...]
    )
    z = logits - jnp.max(logits, axis=1, keepdims=True)
    ez = jnp.exp(z)
    probs = ez / jnp.sum(ez, axis=1, keepdims=True)

    sel = jnp.zeros((TILE, E), jnp.float32)
    lcur = logits
    neg = jnp.float32(-3.0e38)
    for _ in range(TOP_K):
        m = jnp.max(lcur, axis=1, keepdims=True)
        oh = (lcur == m).astype(jnp.float32)
        cs = jnp.dot(oh, trile_ref[...], preferred_element_type=jnp.float32)
        first = oh * (cs == 1.0).astype(jnp.float32)
        sel = sel + first
        lcur = jnp.where(first > 0.0, neg, lcur)

    run = cnt_ref[...]
    cs_tok = jnp.dot(trilt_ref[...], sel, preferred_element_type=jnp.float32)
    keep = ((run + cs_tok - 1.0) < float(CAPACITY)).astype(jnp.float32)
    wgt = probs * sel * keep
    cnt_ref[...] = run + jnp.sum(sel, axis=0, keepdims=True)

    h = (
        jnp.dot(xf, w1c_ref[...], preferred_element_type=jnp.float32)
        + b1c_ref[...]
    )
    hg = h * (0.5 * jax.lax.erf(h * (1.0 / math.sqrt(2.0))) + 0.5)
    we = jnp.dot(wgt, expb_ref[...], preferred_element_type=jnp.float32)
    g = hg * we[:, : E * H]
    acc = jnp.dot(g, w2sp_ref[...], preferred_element_type=jnp.float32)
    out_ref[...] = acc[:, :H] + we[:, E * H :]


@jax.jit
def kernel(x, gate_w, gate_b, w1, b1, w2, b2):
    E, H, D = NUM_EXPERTS, HIDDEN_DIM, INPUT_DIM

    gwt = gate_w.T
    gb = gate_b.reshape(1, E)
    w1c = jnp.transpose(w1, (2, 0, 1)).reshape(D, E * H)
    b1c = b1.reshape(1, E * H)
    w2s = jnp.transpose(w2, (0, 2, 1)).reshape(E * H, H)
    w2sp = jnp.concatenate([w2s, jnp.zeros((E * H, 128 - H), jnp.float32)], axis=1)
    trile = jnp.triu(jnp.ones((E, E), jnp.float32))
    trilt = jnp.tril(jnp.ones((TILE, TILE), jnp.float32))
    expand = jnp.repeat(jnp.eye(E, dtype=jnp.float32), H, axis=1)
    expb = jnp.concatenate([expand, b2], axis=1)

    out = pl.pallas_call(
        _moe_body,
        grid=(NUM_TILES,),
        in_specs=[
            pl.BlockSpec((TILE, D), lambda i: (i, 0)),
            pl.BlockSpec((D, E), lambda i: (0, 0)),
            pl.BlockSpec((1, E), lambda i: (0, 0)),
            pl.BlockSpec((E, E), lambda i: (0, 0)),
            pl.BlockSpec((TILE, TILE), lambda i: (0, 0)),
            pl.BlockSpec((D, E * H), lambda i: (0, 0)),
            pl.BlockSpec((1, E * H), lambda i: (0, 0)),
            pl.BlockSpec((E * H, 128), lambda i: (0, 0)),
            pl.BlockSpec((E, E * H + H), lambda i: (0, 0)),
        ],
        out_specs=pl.BlockSpec((TILE, H), lambda i: (i, 0)),
        out_shape=jax.ShapeDtypeStruct((NUM_TOKENS, H), jnp.float32),
        scratch_shapes=[pltpu.VMEM((1, E), jnp.float32)],
    )(x, gwt, gb, trile, trilt, w1c, b1c, w2sp, expb)

    return out, jnp.float32(0.0)


# speedup vs baseline: 1.1618x; 1.1618x over previous
import math

import jax
import jax.numpy as jnp
from jax.experimental import pallas as pl
from jax.experimental.pallas import tpu as pltpu

NUM_TOKENS = 8192
INPUT_DIM = 1024
NUM_EXPERTS = 64
TOP_K = 8
CAPACITY = 2048
HIDDEN_DIM = 64

TILE = 512
NUM_TILES = NUM_TOKENS // TILE


def _moe_body(
    x_ref, gwt_ref, gb_ref, trile_ref, trilt_ref,
    w1c_ref, b1c_ref, w2sp_ref, expb_ref, out_ref, cnt_ref,
):
    i = pl.program_id(0)

    @pl.when(i == 0)
    def _():
        cnt_ref[...] = jnp.zeros_like(cnt_ref)

    E, H = NUM_EXPERTS, HIDDEN_DIM
    xf = x_ref[...]

    logits = (
        jnp.dot(xf, gwt_ref[...], preferred_element_type=jnp.float32)
        + gb_ref[...]
    )
    z = logits - jnp.max(logits, axis=1, keepdims=True)
    ez = jnp.exp(z)
    probs = ez / jnp.sum(ez, axis=1, keepdims=True)

    sel = jnp.zeros((TILE, E), jnp.float32)
    lcur = logits
    neg = jnp.float32(-3.0e38)
    for _ in range(TOP_K):
        m = jnp.max(lcur, axis=1, keepdims=True)
        oh = (lcur == m).astype(jnp.float32)
        cs = jnp.dot(oh, trile_ref[...], preferred_element_type=jnp.float32)
        first = oh * (cs == 1.0).astype(jnp.float32)
        sel = sel + first
        lcur = jnp.where(first > 0.0, neg, lcur)

    run = cnt_ref[...]
    cs_tok = jnp.dot(trilt_ref[...], sel, preferred_element_type=jnp.float32)
    keep = ((run + cs_tok - 1.0) < float(CAPACITY)).astype(jnp.float32)
    wgt = probs * sel * keep
    cnt_ref[...] = run + jnp.sum(sel, axis=0, keepdims=True)

    h = (
        jnp.dot(
            xf.astype(jnp.bfloat16), w1c_ref[...],
            preferred_element_type=jnp.float32,
        )
        + b1c_ref[...]
    )
    hg = h * (0.5 * jax.lax.erf(h * (1.0 / math.sqrt(2.0))) + 0.5)
    we = jnp.dot(wgt, expb_ref[...], preferred_element_type=jnp.float32)
    g = (hg * we[:, : E * H]).astype(jnp.bfloat16)
    acc = jnp.dot(g, w2sp_ref[...], preferred_element_type=jnp.float32)
    out_ref[...] = acc[:, :H] + we[:, E * H :]


@jax.jit
def kernel(x, gate_w, gate_b, w1, b1, w2, b2):
    E, H, D = NUM_EXPERTS, HIDDEN_DIM, INPUT_DIM

    gwt = gate_w.T
    gb = gate_b.reshape(1, E)
    w1c = jnp.transpose(w1, (2, 0, 1)).reshape(D, E * H).astype(jnp.bfloat16)
    b1c = b1.reshape(1, E * H)
    w2s = jnp.transpose(w2, (0, 2, 1)).reshape(E * H, H)
    w2sp = (
        jnp.concatenate([w2s, jnp.zeros((E * H, 128 - H), jnp.float32)], axis=1)
        .astype(jnp.bfloat16)
    )
    trile = jnp.triu(jnp.ones((E, E), jnp.float32))
    trilt = jnp.tril(jnp.ones((TILE, TILE), jnp.float32))
    expand = jnp.repeat(jnp.eye(E, dtype=jnp.float32), H, axis=1)
    expb = jnp.concatenate([expand, b2], axis=1)

    out = pl.pallas_call(
        _moe_body,
        grid=(NUM_TILES,),
        in_specs=[
            pl.BlockSpec((TILE, D), lambda i: (i, 0)),
            pl.BlockSpec((D, E), lambda i: (0, 0)),
            pl.BlockSpec((1, E), lambda i: (0, 0)),
            pl.BlockSpec((E, E), lambda i: (0, 0)),
            pl.BlockSpec((TILE, TILE), lambda i: (0, 0)),
            pl.BlockSpec((D, E * H), lambda i: (0, 0)),
            pl.BlockSpec((1, E * H), lambda i: (0, 0)),
            pl.BlockSpec((E * H, 128), lambda i: (0, 0)),
            pl.BlockSpec((E, E * H + H), lambda i: (0, 0)),
        ],
        out_specs=pl.BlockSpec((TILE, H), lambda i: (i, 0)),
        out_shape=jax.ShapeDtypeStruct((NUM_TOKENS, H), jnp.float32),
        scratch_shapes=[pltpu.VMEM((1, E), jnp.float32)],
    )(x, gwt, gb, trile, trilt, w1c, b1c, w2sp, expb)

    return out, jnp.float32(0.0)


# speedup vs baseline: 1.2283x; 1.0572x over previous
import math

import jax
import jax.numpy as jnp
from jax.experimental import pallas as pl
from jax.experimental.pallas import tpu as pltpu

NUM_TOKENS = 8192
INPUT_DIM = 1024
NUM_EXPERTS = 64
TOP_K = 8
CAPACITY = 2048
HIDDEN_DIM = 64

TILE = 1024
NUM_TILES = NUM_TOKENS // TILE


def _moe_body(
    x_ref, gwt_ref, gb_ref, trile_ref, trilt_ref,
    w1c_ref, b1c_ref, w2sp_ref, expb_ref, out_ref, cnt_ref,
):
    i = pl.program_id(0)

    @pl.when(i == 0)
    def _():
        cnt_ref[...] = jnp.zeros_like(cnt_ref)

    E, H = NUM_EXPERTS, HIDDEN_DIM
    xf = x_ref[...]

    logits = (
        jnp.dot(xf, gwt_ref[...], preferred_element_type=jnp.float32)
        + gb_ref[...]
    )
    z = logits - jnp.max(logits, axis=1, keepdims=True)
    ez = jnp.exp(z)
    probs = ez / jnp.sum(ez, axis=1, keepdims=True)

    sel = jnp.zeros((TILE, E), jnp.float32)
    lcur = logits
    neg = jnp.float32(-3.0e38)
    for _ in range(TOP_K):
        m = jnp.max(lcur, axis=1, keepdims=True)
        oh = (lcur == m).astype(jnp.float32)
        cs = jnp.dot(oh, trile_ref[...], preferred_element_type=jnp.float32)
        first = oh * (cs == 1.0).astype(jnp.float32)
        sel = sel + first
        lcur = jnp.where(first > 0.0, neg, lcur)

    run = cnt_ref[...]
    cs_tok = jnp.dot(trilt_ref[...], sel, preferred_element_type=jnp.float32)
    keep = ((run + cs_tok - 1.0) < float(CAPACITY)).astype(jnp.float32)
    wgt = probs * sel * keep
    cnt_ref[...] = run + jnp.sum(sel, axis=0, keepdims=True)

    h = (
        jnp.dot(
            xf.astype(jnp.bfloat16), w1c_ref[...],
            preferred_element_type=jnp.float32,
        )
        + b1c_ref[...]
    )
    hg = h * (0.5 * jax.lax.erf(h * (1.0 / math.sqrt(2.0))) + 0.5)
    we = jnp.dot(wgt, expb_ref[...], preferred_element_type=jnp.float32)
    g = (hg * we[:, : E * H]).astype(jnp.bfloat16)
    acc = jnp.dot(g, w2sp_ref[...], preferred_element_type=jnp.float32)
    out_ref[...] = acc[:, :H] + we[:, E * H :]


@jax.jit
def kernel(x, gate_w, gate_b, w1, b1, w2, b2):
    E, H, D = NUM_EXPERTS, HIDDEN_DIM, INPUT_DIM

    gwt = gate_w.T
    gb = gate_b.reshape(1, E)
    w1c = jnp.transpose(w1, (2, 0, 1)).reshape(D, E * H).astype(jnp.bfloat16)
    b1c = b1.reshape(1, E * H)
    w2s = jnp.transpose(w2, (0, 2, 1)).reshape(E * H, H)
    w2sp = (
        jnp.concatenate([w2s, jnp.zeros((E * H, 128 - H), jnp.float32)], axis=1)
        .astype(jnp.bfloat16)
    )
    trile = jnp.triu(jnp.ones((E, E), jnp.float32))
    trilt = jnp.tril(jnp.ones((TILE, TILE), jnp.float32))
    expand = jnp.repeat(jnp.eye(E, dtype=jnp.float32), H, axis=1)
    expb = jnp.concatenate([expand, b2], axis=1)

    out = pl.pallas_call(
        _moe_body,
        grid=(NUM_TILES,),
        in_specs=[
            pl.BlockSpec((TILE, D), lambda i: (i, 0)),
            pl.BlockSpec((D, E), lambda i: (0, 0)),
            pl.BlockSpec((1, E), lambda i: (0, 0)),
            pl.BlockSpec((E, E), lambda i: (0, 0)),
            pl.BlockSpec((TILE, TILE), lambda i: (0, 0)),
            pl.BlockSpec((D, E * H), lambda i: (0, 0)),
            pl.BlockSpec((1, E * H), lambda i: (0, 0)),
            pl.BlockSpec((E * H, 128), lambda i: (0, 0)),
            pl.BlockSpec((E, E * H + H), lambda i: (0, 0)),
        ],
        out_specs=pl.BlockSpec((TILE, H), lambda i: (i, 0)),
        out_shape=jax.ShapeDtypeStruct((NUM_TOKENS, H), jnp.float32),
        scratch_shapes=[pltpu.VMEM((1, E), jnp.float32)],
    )(x, gwt, gb, trile, trilt, w1c, b1c, w2sp, expb)

    return out, jnp.float32(0.0)


# speedup vs baseline: 1.2360x; 1.0063x over previous
import math

import jax
import jax.numpy as jnp
from jax.experimental import pallas as pl
from jax.experimental.pallas import tpu as pltpu

NUM_TOKENS = 8192
INPUT_DIM = 1024
NUM_EXPERTS = 64
TOP_K = 8
CAPACITY = 2048
HIDDEN_DIM = 64

TILE = 1024
NUM_TILES = NUM_TOKENS // TILE


def _moe_body(
    x_ref, gwt_ref, gb_ref, trile_ref, trilt_ref,
    w1c_ref, b1c_ref, w2sp_ref, expb_ref, out_ref, cnt_ref,
):
    i = pl.program_id(0)

    @pl.when(i == 0)
    def _():
        cnt_ref[...] = jnp.zeros_like(cnt_ref)

    E, H = NUM_EXPERTS, HIDDEN_DIM
    xf = x_ref[...]

    logits = (
        jnp.dot(xf, gwt_ref[...], preferred_element_type=jnp.float32)
        + gb_ref[...]
    )
    z = logits - jnp.max(logits, axis=1, keepdims=True)
    ez = jnp.exp(z)
    probs = ez / jnp.sum(ez, axis=1, keepdims=True)

    sel = jnp.zeros((TILE, E), jnp.float32)
    lcur = logits
    neg = jnp.float32(-3.0e38)
    for _ in range(TOP_K):
        m = jnp.max(lcur, axis=1, keepdims=True)
        oh = (lcur == m).astype(jnp.float32)
        cs = jnp.dot(oh.astype(jnp.bfloat16), trile_ref[...], preferred_element_type=jnp.float32)
        first = oh * (cs == 1.0).astype(jnp.float32)
        sel = sel + first
        lcur = jnp.where(first > 0.0, neg, lcur)

    run = cnt_ref[...]
    cs_tok = jnp.dot(trilt_ref[...], sel.astype(jnp.bfloat16), preferred_element_type=jnp.float32)
    keep = ((run + cs_tok - 1.0) < float(CAPACITY)).astype(jnp.float32)
    wgt = probs * sel * keep
    cnt_ref[...] = run + jnp.sum(sel, axis=0, keepdims=True)

    h = (
        jnp.dot(
            xf.astype(jnp.bfloat16), w1c_ref[...],
            preferred_element_type=jnp.float32,
        )
        + b1c_ref[...]
    )
    hg = h * (0.5 * jax.lax.erf(h * (1.0 / math.sqrt(2.0))) + 0.5)
    we = jnp.dot(wgt.astype(jnp.bfloat16), expb_ref[...], preferred_element_type=jnp.float32)
    g = (hg * we[:, : E * H]).astype(jnp.bfloat16)
    acc = jnp.dot(g, w2sp_ref[...], preferred_element_type=jnp.float32)
    out_ref[...] = acc[:, :H] + we[:, E * H :]


@jax.jit
def kernel(x, gate_w, gate_b, w1, b1, w2, b2):
    E, H, D = NUM_EXPERTS, HIDDEN_DIM, INPUT_DIM

    gwt = gate_w.T
    gb = gate_b.reshape(1, E)
    w1c = jnp.transpose(w1, (2, 0, 1)).reshape(D, E * H).astype(jnp.bfloat16)
    b1c = b1.reshape(1, E * H)
    w2s = jnp.transpose(w2, (0, 2, 1)).reshape(E * H, H)
    w2sp = (
        jnp.concatenate([w2s, jnp.zeros((E * H, 128 - H), jnp.float32)], axis=1)
        .astype(jnp.bfloat16)
    )
    trile = jnp.triu(jnp.ones((E, E), jnp.bfloat16))
    trilt = jnp.tril(jnp.ones((TILE, TILE), jnp.bfloat16))
    expand = jnp.repeat(jnp.eye(E, dtype=jnp.bfloat16), H, axis=1)
    expb = jnp.concatenate([expand, b2.astype(jnp.bfloat16)], axis=1)

    out = pl.pallas_call(
        _moe_body,
        grid=(NUM_TILES,),
        in_specs=[
            pl.BlockSpec((TILE, D), lambda i: (i, 0)),
            pl.BlockSpec((D, E), lambda i: (0, 0)),
            pl.BlockSpec((1, E), lambda i: (0, 0)),
            pl.BlockSpec((E, E), lambda i: (0, 0)),
            pl.BlockSpec((TILE, TILE), lambda i: (0, 0)),
            pl.BlockSpec((D, E * H), lambda i: (0, 0)),
            pl.BlockSpec((1, E * H), lambda i: (0, 0)),
            pl.BlockSpec((E * H, 128), lambda i: (0, 0)),
            pl.BlockSpec((E, E * H + H), lambda i: (0, 0)),
        ],
        out_specs=pl.BlockSpec((TILE, H), lambda i: (i, 0)),
        out_shape=jax.ShapeDtypeStruct((NUM_TOKENS, H), jnp.float32),
        scratch_shapes=[pltpu.VMEM((1, E), jnp.float32)],
    )(x, gwt, gb, trile, trilt, w1c, b1c, w2sp, expb)

    return out, jnp.float32(0.0)
